# Initial kernel scaffold; baseline (speedup 1.0000x reference)
import jax
import jax.numpy as jnp
from jax.experimental import pallas as pl


def kernel(x, edge_index, W1, b1, att1, bias1, W2, b2, att2, bias2):
    raise NotImplementedError("write your pallas kernel here")


# speedup vs baseline: 1.9802x; 1.9802x over previous
import jax
import jax.numpy as jnp
from jax import lax
from jax.experimental import pallas as pl
from jax.experimental.pallas import tpu as pltpu
from jax.experimental.pallas import tpu_sc as plsc

N = 10000
E = 320000
D_IN = 128
HID = 128
HEADS = 4
C = 40
NS = 0.2
RWH = 32
HH = 16
QN = 8
RW2 = 48
L2W = 16
L2H = 8
L2Q = 5
R = 400
GRID = N // R
KB = 80
NSUB = 16
NPAD = 10240
STRIPE = NPAD // NSUB
ZROWS = 64
AUXW = 2200000
EPADW = 1100000


def _lrelu(v):
    return jnp.where(v >= 0, v, NS * v)


def _ka_body(x_ref, w1_ref, b1_ref, acat_ref, fe_ref, s8_ref, m_ref):
    i = pl.program_id(0)
    fea = jnp.dot(x_ref[...], w1_ref[...],
                  preferred_element_type=jnp.float32) + b1_ref[...]
    s8 = jnp.dot(fea, acat_ref[...], preferred_element_type=jnp.float32)
    s8_ref[...] = s8
    feah = fea.reshape(R, HEADS, QN, HH)
    ones = jnp.ones((R, HEADS, QN, 1), jnp.float32)
    pad = jnp.zeros((R, HEADS, QN, RWH - HH - 1), jnp.float32)
    fe_ref[...] = jnp.concatenate([feah, ones, pad], axis=-1)
    bm = jnp.max(s8, axis=0, keepdims=True)

    @pl.when(i == 0)
    def _():
        m_ref[...] = bm

    @pl.when(i > 0)
    def _():
        m_ref[...] = jnp.maximum(m_ref[...], bm)


def _run_a(x, W1, b1r, acat):
    return pl.pallas_call(
        _ka_body,
        grid=(GRID,),
        in_specs=[
            pl.BlockSpec((R, D_IN), lambda i: (i, 0)),
            pl.BlockSpec((D_IN, HEADS * HID), lambda i: (0, 0)),
            pl.BlockSpec((1, HEADS * HID), lambda i: (0, 0)),
            pl.BlockSpec((HEADS * HID, 8), lambda i: (0, 0)),
        ],
        out_specs=[
            pl.BlockSpec((R, HEADS, QN, RWH), lambda i: (i, 0, 0, 0)),
            pl.BlockSpec((R, 8), lambda i: (i, 0)),
            pl.BlockSpec((1, 8), lambda i: (0, 0)),
        ],
        out_shape=[
            jax.ShapeDtypeStruct((N, HEADS, QN, RWH), jnp.float32),
            jax.ShapeDtypeStruct((N, 8), jnp.float32),
            jax.ShapeDtypeStruct((1, 8), jnp.float32),
        ],
    )(x, W1, b1r, acat)


def _kb_body(edge_ref, fea_ref, aux_ref, z_ref, out_ref,
             s8_v, m_v, si_v, di_v, gi_v, w_all, rows_v, acc_sh, sem):
    c = lax.axis_index("c")
    s = lax.axis_index("s")
    ept = E // NSUB
    nblk = ept // KB
    pltpu.sync_copy(aux_ref.at[pl.ds(0, N * 8)], s8_v)
    pltpu.sync_copy(aux_ref.at[pl.ds(N * 8, 8)], m_v)

    for hh in range(2):
        h = c * 2 + hh
        hv = jnp.full((16,), 0, jnp.int32) + h
        mh = plsc.load_gather(m_v, [jnp.full((16,), 4, jnp.int32) + h])

        def wblk(b, carry):
            off = s * ept + b * KB
            pltpu.sync_copy(edge_ref.at[0, pl.ds(off, KB)], si_v)
            pltpu.sync_copy(edge_ref.at[1, pl.ds(off, KB)], di_v)
            for g in range(KB // 16):
                si = si_v[pl.ds(g * 16, 16)]
                di = di_v[pl.ds(g * 16, 16)]
                sd = plsc.load_gather(s8_v, [di * 8 + hv])
                ss = plsc.load_gather(s8_v, [si * 8 + hv + 4])
                a = _lrelu(sd + ss)
                bd = _lrelu(sd + mh)
                w = jnp.exp(a - bd)
                w = jnp.where(si == di, 0.0, w)
                w_all[pl.ds(b * KB + g * 16, 16)] = w
            return carry

        lax.fori_loop(0, nblk, wblk, 0)

        def tpass(t, tcarry):
            pltpu.sync_copy(z_ref, acc_sh.at[pl.ds(s * STRIPE, STRIPE)])
            plsc.subcore_barrier()

            def blk(b, carry):
                off = s * ept + b * KB
                pltpu.sync_copy(edge_ref.at[0, pl.ds(off, KB)], si_v)
                pltpu.sync_copy(edge_ref.at[1, pl.ds(off, KB)], di_v)
                for g in range(KB // 16):
                    si = si_v[pl.ds(g * 16, 16)]
                    gi_v[pl.ds(g * 16, 16)] = si * (HEADS * QN) + h * QN + t
                pltpu.async_copy(fea_ref.at[gi_v], rows_v, sem).wait()
                for e in range(KB):
                    wb = plsc.load_gather(
                        w_all, [jnp.full((16,), 0, jnp.int32) + (b * KB + e)])
                    for r in range(RWH // 16):
                        rows_v[e, pl.ds(r * 16, 16)] = (
                            rows_v[e, pl.ds(r * 16, 16)] * wb)
                pltpu.sync_copy(rows_v, acc_sh.at[di_v], add=True)
                return carry

            lax.fori_loop(0, nblk, blk, 0)
            plsc.subcore_barrier()
            pltpu.sync_copy(acc_sh.at[pl.ds(s * STRIPE, STRIPE)],
                            out_ref.at[pl.ds(s * STRIPE, STRIPE), h, t])
            return tcarry

        lax.fori_loop(0, QN, tpass, 0)


def _run_b(edge_index, fea_flat, aux1, zeros1):
    mesh = plsc.VectorSubcoreMesh(core_axis_name="c", subcore_axis_name="s")
    kb = pl.kernel(
        _kb_body,
        out_type=jax.ShapeDtypeStruct((NPAD, HEADS, QN, RWH), jnp.float32),
        mesh=mesh,
        scratch_types=[
            pltpu.VMEM((N * 8,), jnp.float32),
            pltpu.VMEM((8,), jnp.float32),
            pltpu.VMEM((KB,), jnp.int32),
            pltpu.VMEM((KB,), jnp.int32),
            pltpu.VMEM((KB,), jnp.int32),
            pltpu.VMEM((E // NSUB,), jnp.float32),
            pltpu.VMEM((KB, RWH), jnp.float32),
            pltpu.VMEM_SHARED((NPAD, RWH), jnp.float32),
            pltpu.SemaphoreType.DMA,
        ],
        compiler_params=pltpu.CompilerParams(
            use_tc_tiling_on_sc=False, needs_layout_passes=False),
    )
    return kb(edge_index, fea_flat, aux1, zeros1)


def _kc_body(acc_ref, fe_ref, s8_ref, m_ref, bias1_ref, w2_ref, b2_ref,
             a2_ref, fe2_ref, s2_ref, m2_ref):
    i = pl.program_id(0)
    acc = acc_ref[...]
    num = jnp.concatenate([acc[:, :, q, :HH] for q in range(QN)], axis=-1)
    den = acc[:, :, 0, HH]
    s8 = s8_ref[...]
    sd, ss = s8[:, :HEADS], s8[:, HEADS:]
    m = m_ref[...][:, HEADS:]
    w_self = jnp.exp(_lrelu(sd + ss) - _lrelu(sd + m))
    fe = fe_ref[...]
    feah = jnp.concatenate([fe[:, :, q, :HH] for q in range(QN)], axis=-1)
    numt = num + w_self[:, :, None] * feah
    dent = den + w_self
    h = (numt / dent[:, :, None]).reshape(R, HEADS * HID) + bias1_ref[...]
    h = jnp.maximum(h, 0.0)
    fea2 = jnp.dot(h, w2_ref[...], preferred_element_type=jnp.float32) \
        + b2_ref[...]
    s2 = jnp.dot(fea2, a2_ref[...], preferred_element_type=jnp.float32)
    s2_ref[...] = jnp.concatenate(
        [s2, jnp.zeros((R, 8 - 2), jnp.float32)], axis=1)
    ones1 = jnp.ones((R, 1), jnp.float32)
    zpad = jnp.zeros((R, L2W - L2H - 1), jnp.float32)
    fe2_ref[...] = jnp.concatenate(
        [jnp.concatenate([fea2[:, q * L2H:(q + 1) * L2H], ones1, zpad],
                         axis=1) for q in range(L2Q)],
        axis=1).reshape(R, L2Q, L2W)
    bm = jnp.max(s2_ref[...], axis=0, keepdims=True)

    @pl.when(i == 0)
    def _():
        m2_ref[...] = bm

    @pl.when(i > 0)
    def _():
        m2_ref[...] = jnp.maximum(m2_ref[...], bm)


def _run_c(acc1, fea_ext, s8, m8, bias1r, W2p, b2p, A2):
    return pl.pallas_call(
        _kc_body,
        grid=(GRID,),
        in_specs=[
            pl.BlockSpec((R, HEADS, QN, RWH), lambda i: (i, 0, 0, 0)),
            pl.BlockSpec((R, HEADS, QN, RWH), lambda i: (i, 0, 0, 0)),
            pl.BlockSpec((R, 8), lambda i: (i, 0)),
            pl.BlockSpec((1, 8), lambda i: (0, 0)),
            pl.BlockSpec((1, HEADS * HID), lambda i: (0, 0)),
            pl.BlockSpec((HEADS * HID, RW2), lambda i: (0, 0)),
            pl.BlockSpec((1, RW2), lambda i: (0, 0)),
            pl.BlockSpec((RW2, 2), lambda i: (0, 0)),
        ],
        out_specs=[
            pl.BlockSpec((R, L2Q, L2W), lambda i: (i, 0, 0)),
            pl.BlockSpec((R, 8), lambda i: (i, 0)),
            pl.BlockSpec((1, 8), lambda i: (0, 0)),
        ],
        out_shape=[
            jax.ShapeDtypeStruct((N, L2Q, L2W), jnp.float32),
            jax.ShapeDtypeStruct((N, 8), jnp.float32),
            jax.ShapeDtypeStruct((1, 8), jnp.float32),
        ],
    )(acc1, fea_ext, s8, m8, bias1r, W2p, b2p, A2)


def _kd_body(edge_ref, fe2_ref, aux_ref, z_ref, out_ref,
             s2_v, m_v, si_v, di_v, gi_v, w_all, rows_v, acc2_sh, sem):
    c = lax.axis_index("c")
    s = lax.axis_index("s")
    ept = E // (2 * NSUB)
    nblk = ept // KB
    pltpu.sync_copy(aux_ref.at[pl.ds(0, N * 8)], s2_v)
    pltpu.sync_copy(aux_ref.at[pl.ds(N * 8, 8)], m_v)
    zv = jnp.full((16,), 0, jnp.int32)
    mh = plsc.load_gather(m_v, [zv + 1])

    for t in range(L2Q):
        pltpu.sync_copy(z_ref, acc2_sh.at[pl.ds(s * STRIPE, STRIPE)])
        plsc.subcore_barrier()

        def blk(b, carry):
            off = (c * NSUB + s) * ept + b * KB
            pltpu.sync_copy(edge_ref.at[0, pl.ds(off, KB)], si_v)
            pltpu.sync_copy(edge_ref.at[1, pl.ds(off, KB)], di_v)
            for g in range(KB // 16):
                si = si_v[pl.ds(g * 16, 16)]
                if t == 0:
                    di = di_v[pl.ds(g * 16, 16)]
                    sd = plsc.load_gather(s2_v, [di * 8 + zv])
                    ss = plsc.load_gather(s2_v, [si * 8 + zv + 1])
                    a = _lrelu(sd + ss)
                    bd = _lrelu(sd + mh)
                    w = jnp.exp(a - bd)
                    w = jnp.where(si == di, 0.0, w)
                    w_all[pl.ds(b * KB + g * 16, 16)] = w
                gi_v[pl.ds(g * 16, 16)] = si * L2Q + t
            pltpu.async_copy(fe2_ref.at[gi_v], rows_v, sem).wait()
            for e in range(KB):
                wb = plsc.load_gather(
                    w_all, [jnp.full((16,), 0, jnp.int32) + (b * KB + e)])
                rows_v[e, pl.ds(0, 16)] = rows_v[e, pl.ds(0, 16)] * wb
            pltpu.sync_copy(rows_v, acc2_sh.at[di_v], add=True)
            return carry

        lax.fori_loop(0, nblk, blk, 0)
        plsc.subcore_barrier()
        pltpu.sync_copy(acc2_sh.at[pl.ds(s * STRIPE, STRIPE)],
                        out_ref.at[pl.ds(s * STRIPE, STRIPE), c, t])


def _run_d(edge_index, fe2_flat, aux2, zeros2):
    mesh = plsc.VectorSubcoreMesh(core_axis_name="c", subcore_axis_name="s")
    kd = pl.kernel(
        _kd_body,
        out_type=jax.ShapeDtypeStruct((NPAD, 2, L2Q, L2W), jnp.float32),
        mesh=mesh,
        scratch_types=[
            pltpu.VMEM((N * 8,), jnp.float32),
            pltpu.VMEM((8,), jnp.float32),
            pltpu.VMEM((KB,), jnp.int32),
            pltpu.VMEM((KB,), jnp.int32),
            pltpu.VMEM((KB,), jnp.int32),
            pltpu.VMEM((E // (2 * NSUB),), jnp.float32),
            pltpu.VMEM((KB, L2W), jnp.float32),
            pltpu.VMEM_SHARED((NPAD, L2W), jnp.float32),
            pltpu.SemaphoreType.DMA,
        ],
        compiler_params=pltpu.CompilerParams(
            use_tc_tiling_on_sc=False, needs_layout_passes=False),
    )
    return kd(edge_index, fe2_flat, aux2, zeros2)


def _ke_body(acc_ref, fe2_ref, s2_ref, m2_ref, bias2_ref, out_ref):
    acc = acc_ref[...]
    tot = acc[:, 0] + acc[:, 1]
    num = jnp.concatenate([tot[:, q, :L2H] for q in range(L2Q)], axis=1)
    den = tot[:, 0, L2H:L2H + 1]
    s2 = s2_ref[...]
    sd, ss = s2[:, 0:1], s2[:, 1:2]
    m2 = m2_ref[...][:, 1:2]
    w_self = jnp.exp(_lrelu(sd + ss) - _lrelu(sd + m2))
    fe2 = fe2_ref[...]
    fe2c = jnp.concatenate([fe2[:, q, :L2H] for q in range(L2Q)], axis=1)
    numt = num + w_self * fe2c
    dent = den + w_self
    logits = numt / dent + bias2_ref[...][:, :C]
    mx = jnp.max(logits, axis=1, keepdims=True)
    ex = jnp.exp(logits - mx)
    out_ref[...] = ex / jnp.sum(ex, axis=1, keepdims=True)


def _run_e(acc2, fe2, s2, m2, bias2r):
    return pl.pallas_call(
        _ke_body,
        grid=(GRID,),
        in_specs=[
            pl.BlockSpec((R, 2, L2Q, L2W), lambda i: (i, 0, 0, 0)),
            pl.BlockSpec((R, L2Q, L2W), lambda i: (i, 0, 0)),
            pl.BlockSpec((R, 8), lambda i: (i, 0)),
            pl.BlockSpec((1, 8), lambda i: (0, 0)),
            pl.BlockSpec((1, RW2), lambda i: (0, 0)),
        ],
        out_specs=pl.BlockSpec((R, C), lambda i: (i, 0)),
        out_shape=jax.ShapeDtypeStruct((N, C), jnp.float32),
    )(acc2, fe2, s2, m2, bias2r)


def kernel(x, edge_index, W1, b1, att1, bias1, W2, b2, att2, bias2):
    f32 = jnp.float32
    att_i = att1[0, :, :HID]
    att_j = att1[0, :, HID:]
    eye = jnp.eye(HEADS, dtype=f32)
    Ai = (att_i[:, :, None] * eye[:, None, :]).reshape(HEADS * HID, HEADS)
    Aj = (att_j[:, :, None] * eye[:, None, :]).reshape(HEADS * HID, HEADS)
    acat = jnp.concatenate([Ai, Aj], axis=1)
    b1r = b1.reshape(1, HEADS * HID)
    bias1r = bias1.reshape(1, HEADS * HID)
    W2p = jnp.concatenate(
        [W2, jnp.zeros((HEADS * HID, RW2 - C), f32)], axis=1)
    b2p = jnp.concatenate([b2, jnp.zeros((RW2 - C,), f32)]).reshape(1, RW2)
    A2 = jnp.concatenate([
        jnp.concatenate([att2[0, 0, :C, None], att2[0, 0, C:, None]], axis=1),
        jnp.zeros((RW2 - C, 2), f32)], axis=0)
    bias2r = jnp.concatenate(
        [bias2, jnp.zeros((RW2 - C,), f32)]).reshape(1, RW2)

    edge_pad = jnp.concatenate(
        [edge_index, jnp.zeros((2, EPADW - E), jnp.int32)], axis=1)
    zeros1 = jnp.zeros((STRIPE, RWH), f32)
    zeros2 = jnp.zeros((STRIPE, L2W), f32)
    fea_ext, s8, m8raw = _run_a(x, W1, b1r, acat)
    fea_flat = fea_ext.reshape(N * HEADS * QN, RWH)
    auxpad = jnp.zeros((AUXW - N * 8 - 8,), f32)
    aux1 = jnp.concatenate([s8.reshape(-1), m8raw.reshape(-1), auxpad])
    acc1 = _run_b(edge_pad, fea_flat, aux1, zeros1)
    fe2, s2, m28raw = _run_c(acc1, fea_ext, s8, m8raw, bias1r, W2p, b2p, A2)
    fe2_flat = fe2.reshape(N * L2Q, L2W)
    aux2 = jnp.concatenate([s2.reshape(-1), m28raw.reshape(-1), auxpad])
    acc2 = _run_d(edge_pad, fe2_flat, aux2, zeros2)
    return _run_e(acc2, fe2, s2, m28raw, bias2r)
